# baseline (speedup 1.0000x reference)
import jax
import jax.numpy as jnp
from jax import lax
from jax.experimental import pallas as pl
from jax.experimental.pallas import tpu as pltpu
from jax.experimental.pallas import tpu_sc as plsc
from jax.experimental.layout import Layout, with_layout_constraint

NUM_FACTORS = 64
BATCH = 16384
NC, NS, L = 2, 16, 16
NW = NC * NS
B_PER_W = BATCH // NW
NG = B_PER_W // L
NK = NUM_FACTORS // L
EIGHTH = 64
PH = 8


def _body(user_idx, item_i_idx, feature_i_idx, item_j_idx, feature_j_idx,
          user_table, item_table, visual_table, visual_bias_table,
          out_hbm,
          iu, ii, ifi, ij, ifj,
          bu0, bti0, btj0, bvi0, bvj0,
          bu1, bti1, btj1, bvi1, bvj1,
          bi_buf, bj_buf, obuf, bsem, rsem0, rsem1):
    wid = lax.axis_index("s") * NC + lax.axis_index("c")
    base = wid * B_PER_W
    lane = lax.iota(jnp.int32, L)

    idxs = (iu, ii, ij, ifi, ifj)
    srcs = (user_idx, item_i_idx, item_j_idx, feature_i_idx, feature_j_idx)
    tabs = (user_table, item_table, item_table, visual_table, visual_table)
    blks0 = (bu0, bti0, btj0, bvi0, bvj0)
    blks1 = (bu1, bti1, btj1, bvi1, bvj1)

    def restage(hoff, e8):
        off = hoff + e8 * EIGHTH
        for src, idxb in zip(srcs, idxs):
            pltpu.sync_copy(src.at[pl.ds(off, EIGHTH)], idxb)
        cpa = pltpu.async_copy(visual_bias_table.at[ifi], bi_buf, bsem)
        cpb = pltpu.async_copy(visual_bias_table.at[ifj], bj_buf, bsem)
        cpa.wait()
        cpb.wait()

    def fire(g, par, blks, sem):
        w = jnp.bitwise_and(g, 3) * L
        vecs = [idxb[pl.ds(w, L)] for idxb in idxs]
        for tab, blk, vec in zip(tabs, blks, vecs):
            for jj in range(PH):
                q = vec[jj + par * PH]
                blk8 = pl.multiple_of(
                    lax.shift_left(lax.shift_right_logical(q, 3), 3), 8)
                pltpu.async_copy(tab.at[pl.ds(blk8, 8), :], blk.at[jj], sem)
        return tuple(vecs)

    def drain(blks, sem):
        for tab, blk in zip(tabs, blks):
            for jj in range(PH):
                pltpu.make_async_copy(tab.at[pl.ds(0, 8), :],
                                      blk.at[jj], sem).wait()

    def compute(vecs, par, blks, acc):
        for jj in range(PH):
            r = jj + par * PH
            subs = [jnp.bitwise_and(v[r], 7) for v in vecs]
            p = jnp.zeros((L,), jnp.float32)
            for k in range(NK):
                sl = pl.ds(k * L, L)
                comb = (blks[1][jj, subs[1], sl] - blks[2][jj, subs[2], sl]
                        + blks[3][jj, subs[3], sl] - blks[4][jj, subs[4], sl])
                p = p + blks[0][jj, subs[0], sl] * comb
            for s in (8, 4, 2, 1):
                p = p + p[jnp.bitwise_xor(lane, s)]
            acc = jnp.where(lane == r, p + acc, acc)
        return acc

    restage(base, 0)
    vecs0 = fire(0, 0, blks0, rsem0)

    def body(g, vecs_a):
        w = jnp.bitwise_and(g, 3) * L
        acc = bi_buf[pl.ds(w, L)] - bj_buf[pl.ds(w, L)]
        vecs_b = fire(g, 1, blks1, rsem1)
        drain(blks0, rsem0)
        acc = compute(vecs_a, 0, blks0, acc)
        g2 = jnp.minimum(g + 1, NG - 1)

        @pl.when(jnp.bitwise_and(g2, 3) == 0)
        def _():
            restage(base, lax.shift_right_logical(g2, 2))

        vecs_n = fire(g2, 0, blks0, rsem0)
        drain(blks1, rsem1)
        acc = compute(vecs_b, 1, blks1, acc)
        obuf[pl.ds(0, L)] = acc
        pltpu.sync_copy(obuf, out_hbm.at[pl.ds(base + g * L, L)])
        return vecs_n

    lax.fori_loop(0, NG, body, vecs0)
    drain(blks0, rsem0)


@jax.jit
def _run(user, item_i, feature_i, item_j, feature_j,
         user_table, item_table, visual_table, visual_bias_table):
    mesh = plsc.VectorSubcoreMesh(core_axis_name="c", subcore_axis_name="s")
    grid_kernel = pl.kernel(
        _body,
        out_type=jax.ShapeDtypeStruct((BATCH,), jnp.float32),
        mesh=mesh,
        scratch_types=(
            [pltpu.VMEM((EIGHTH,), jnp.int32) for _ in range(5)]
            + [pltpu.VMEM((PH, 8, NUM_FACTORS), jnp.float32)
               for _ in range(10)]
            + [pltpu.VMEM((EIGHTH,), jnp.float32) for _ in range(2)]
            + [pltpu.VMEM((L,), jnp.float32),
               pltpu.SemaphoreType.DMA,
               pltpu.SemaphoreType.DMA,
               pltpu.SemaphoreType.DMA]
        ),
    )
    fmt = Layout(major_to_minor=(0, 1))
    user_table, item_table, visual_table = lax.optimization_barrier(
        tuple(with_layout_constraint(t, fmt)
              for t in (user_table, item_table, visual_table)))
    return grid_kernel(user, item_i, feature_i, item_j, feature_j,
                       user_table, item_table, visual_table,
                       visual_bias_table)


def kernel(user, item_i, feature_i, item_j, feature_j,
           user_table, item_table, visual_table, visual_bias_table):
    return _run(user.astype(jnp.int32), item_i.astype(jnp.int32),
                feature_i.astype(jnp.int32), item_j.astype(jnp.int32),
                feature_j.astype(jnp.int32),
                user_table, item_table, visual_table,
                visual_bias_table.reshape(-1))
